# baseline (speedup 1.0000x reference)
import functools

import jax
import jax.numpy as jnp
from jax import lax
from jax.experimental import pallas as pl
from jax.experimental.pallas import tpu as pltpu
from jax.experimental.pallas import tpu_sc as plsc

_NC = 2
_NS = 16
_NW = _NC * _NS

_IDX_PER_DMA = 128
_KB = 4
_CHUNK = _IDX_PER_DMA * _KB


def _sc_gather(x_flat, table):
    n = x_flat.shape[0]
    d = table.shape[1]
    rows_per_w = n // _NW
    nchunk = rows_per_w // _CHUNK
    assert rows_per_w % _CHUNK == 0 and nchunk % 2 == 0

    x_view = x_flat.reshape(_NW, nchunk, _KB, _IDX_PER_DMA)

    mesh = plsc.VectorSubcoreMesh(core_axis_name="c", subcore_axis_name="s")

    @functools.partial(
        pl.kernel,
        out_type=jax.ShapeDtypeStruct((_NW, nchunk, _CHUNK, d), jnp.float32),
        mesh=mesh,
        scratch_types=[
            pltpu.VMEM((2, _KB, _IDX_PER_DMA), jnp.int32),
            pltpu.VMEM((2, _CHUNK, d), jnp.float32),
            pltpu.SemaphoreType.DMA,
            pltpu.SemaphoreType.DMA,
        ],
        compiler_params=pltpu.CompilerParams(use_tc_tiling_on_sc=False),
    )
    def gather_kernel(x_hbm, table_hbm, e_hbm, idx_v, rows_v, sem0, sem1):
        wid = lax.axis_index("s") * _NC + lax.axis_index("c")
        sems = (sem0, sem1)

        def body(i, carry):
            handles = []
            for s in range(2):
                chunk = i * 2 + s
                pltpu.sync_copy(x_hbm.at[wid, chunk], idx_v.at[s])
                hs = [
                    pltpu.async_copy(
                        table_hbm.at[idx_v.at[s, j]],
                        rows_v.at[s, pl.ds(j * _IDX_PER_DMA, _IDX_PER_DMA)],
                        sems[s],
                    )
                    for j in range(_KB)
                ]
                handles.append(hs)
            for s in range(2):
                for h in handles[s]:
                    h.wait()
                pltpu.sync_copy(rows_v.at[s], e_hbm.at[wid, i * 2 + s])
            return carry

        lax.fori_loop(0, nchunk // 2, body, 0)

    return gather_kernel(x_view, table).reshape(n, d)


def _tc_project(e, pe, wt, b2d):
    bsz, t, d = e.shape
    m = wt.shape[1]
    bb = 64
    assert bsz % bb == 0

    def mm_kernel(e_ref, pe_ref, w_ref, b_ref, out_ref):
        eb = e_ref[...] + pe_ref[...][None, :, :]
        acc = jax.lax.dot_general(
            eb.reshape(bb * t, d),
            w_ref[...],
            (((1,), (0,)), ((), ())),
            preferred_element_type=jnp.float32,
        )
        out_ref[...] = (acc + b_ref[...]).reshape(bb, t, m)

    return pl.pallas_call(
        mm_kernel,
        grid=(bsz // bb,),
        in_specs=[
            pl.BlockSpec((bb, t, d), lambda i: (i, 0, 0)),
            pl.BlockSpec((t, d), lambda i: (0, 0)),
            pl.BlockSpec((d, m), lambda i: (0, 0)),
            pl.BlockSpec((1, m), lambda i: (0, 0)),
        ],
        out_specs=pl.BlockSpec((bb, t, m), lambda i: (i, 0, 0)),
        out_shape=jax.ShapeDtypeStruct((bsz, t, m), jnp.float32),
    )(e, pe, wt, b2d)


def kernel(x, table, pe, W, b):
    if x.ndim == 1:
        x = x[None, :]
    bsz, t = x.shape
    e_flat = _sc_gather(x.reshape(-1), table)
    e = e_flat.reshape(bsz, t, table.shape[1])
    return _tc_project(e, pe, W.T, b[None, :])
